# baseline (speedup 1.0000x reference)
import functools

import jax
import jax.numpy as jnp
from jax import lax
from jax.experimental import pallas as pl
from jax.experimental.pallas import tpu as pltpu
from jax.experimental.pallas import tpu_sc as plsc

MAX_V = 1000
VPAD = 1024
D = 128
B_TOT = 1024 * 50
NW = 32
B_PER_W = B_TOT // NW
CHUNK = 80
NCHUNK = B_PER_W // CHUNK


def _table_body(w_ref, b_ref, o_ref):
    o_ref[...] = w_ref[...].T + b_ref[...]


def _build_table(w_pad, b_row):
    return pl.pallas_call(
        _table_body,
        out_shape=jax.ShapeDtypeStruct((VPAD, D), jnp.float32),
    )(w_pad, b_row)


_mesh = plsc.VectorSubcoreMesh(core_axis_name="c", subcore_axis_name="s")


@functools.partial(
    pl.kernel,
    mesh=_mesh,
    out_type=jax.ShapeDtypeStruct((B_TOT, D), jnp.float32),
    scratch_types=[
        pltpu.VMEM((NCHUNK, CHUNK), jnp.int32),
        pltpu.VMEM((CHUNK, D), jnp.float32),
        pltpu.VMEM((CHUNK, D), jnp.float32),
        pltpu.SemaphoreType.DMA,
        pltpu.SemaphoreType.DMA,
    ],
)
def _sc_gather(table_hbm, idx_hbm, out_hbm, idx_v, rows0, rows1, gsem, ssem):
    wid = lax.axis_index("s") * 2 + lax.axis_index("c")
    base = wid * B_PER_W
    pltpu.sync_copy(idx_hbm.at[wid], idx_v)
    bufs = (rows0, rows1)
    pending = [None, None]
    for j in range(NCHUNK):
        k = j % 2
        if pending[k] is not None:
            pending[k].wait()
        pltpu.async_copy(table_hbm.at[idx_v.at[j]], bufs[k], gsem).wait()
        pending[k] = pltpu.async_copy(
            bufs[k], out_hbm.at[pl.ds(base + j * CHUNK, CHUNK)], ssem
        )
    pending[0].wait()
    pending[1].wait()


def kernel(x, W, b):
    w_pad = jnp.pad(W, ((0, 0), (0, VPAD - MAX_V)))
    table = _build_table(w_pad, b.reshape(1, D))
    idx = x.reshape(NW, NCHUNK, CHUNK).astype(jnp.int32)
    out = _sc_gather(table, idx)
    return out.reshape(x.shape[0], x.shape[1], D)


# speedup vs baseline: 1.7931x; 1.7931x over previous
import functools

import jax
import jax.numpy as jnp
from jax import lax
from jax.experimental import pallas as pl
from jax.experimental.pallas import tpu as pltpu
from jax.experimental.pallas import tpu_sc as plsc

MAX_V = 1000
VPAD = 1024
D = 128
NB = 1024
L = 50
NW = 32
RPW = NB // NW


def _table_body(w_ref, b_ref, o_ref):
    o_ref[...] = w_ref[...].T + b_ref[...]


def _build_table(w_pad, b_row):
    return pl.pallas_call(
        _table_body,
        out_shape=jax.ShapeDtypeStruct((VPAD, D), jnp.float32),
    )(w_pad, b_row)


_mesh = plsc.VectorSubcoreMesh(core_axis_name="c", subcore_axis_name="s")


@functools.partial(
    pl.kernel,
    mesh=_mesh,
    out_type=jax.ShapeDtypeStruct((NB, L, D), jnp.float32),
    scratch_types=[
        pltpu.VMEM((RPW, L), jnp.int32),
        pltpu.VMEM((2, L, D), jnp.float32),
        pltpu.VMEM_SHARED((VPAD, D), jnp.float32),
        pltpu.SemaphoreType.DMA,
    ],
)
def _sc_emb(table_hbm, x_hbm, out_hbm, idx_v, buf, tbl_s, gsem):
    c = lax.axis_index("c")
    s = lax.axis_index("s")
    wid = s * 2 + c
    r0 = wid * RPW
    pltpu.sync_copy(x_hbm.at[pl.ds(r0, RPW)], idx_v)
    @pl.when(s == 0)
    def _():
        pltpu.sync_copy(table_hbm, tbl_s)
    plsc.subcore_barrier()

    def step(m, carry):
        i = 2 * m
        cp0 = pltpu.async_copy(tbl_s.at[idx_v.at[i]], buf.at[0], gsem)
        cp1 = pltpu.async_copy(tbl_s.at[idx_v.at[i + 1]], buf.at[1], gsem)
        cp0.wait()
        cp1.wait()
        pltpu.sync_copy(buf, out_hbm.at[pl.ds(r0 + i, 2)])
        return carry

    lax.fori_loop(0, RPW // 2, step, 0)


def kernel(x, W, b):
    w_pad = jnp.pad(W, ((0, 0), (0, VPAD - MAX_V)))
    table = _build_table(w_pad, b.reshape(1, D))
    return _sc_emb(table, x.astype(jnp.int32))
